# Initial kernel scaffold; baseline (speedup 1.0000x reference)
import jax
import jax.numpy as jnp
from jax.experimental import pallas as pl


def kernel(relative, flatten_index):
    raise NotImplementedError("write your pallas kernel here")


# speedup vs baseline: 23.7797x; 23.7797x over previous
import functools

import jax
import jax.numpy as jnp
from jax import lax
from jax.experimental import pallas as pl
from jax.experimental.pallas import tpu as pltpu
from jax.experimental.pallas import tpu_sc as plsc

DIM = 2048
N_CHANNELS = 32
TABLE = 2 * DIM - 1
PADS = 16
WIDTH = TABLE + PADS + 1
N_Q = 8
N_K = 8
N_V = 16
K_INFLIGHT = 8


def _emit_rows(dst_hbm, c_local, buf, sem):
    def body(x, carry):
        s = (DIM - 1) - x
        a = (s + (PADS - 1)) & (-PADS)
        a = pl.multiple_of(a, PADS)
        p = a - s
        pltpu.make_async_copy(
            buf.at[p, pl.ds(a, DIM)], dst_hbm.at[c_local, x], sem
        ).start()

        @pl.when(x >= K_INFLIGHT)
        def _():
            pltpu.make_async_copy(
                buf.at[0, pl.ds(0, DIM)], dst_hbm.at[c_local, x], sem
            ).wait()

        return carry

    lax.fori_loop(0, DIM, body, 0)

    def drain(i, carry):
        pltpu.make_async_copy(
            buf.at[0, pl.ds(0, DIM)], dst_hbm.at[c_local, 0], sem
        ).wait()
        return carry

    lax.fori_loop(0, K_INFLIGHT, drain, 0)


@functools.partial(
    pl.kernel,
    out_type=(
        jax.ShapeDtypeStruct((N_Q, DIM, DIM), jnp.float32),
        jax.ShapeDtypeStruct((N_K, DIM, DIM), jnp.float32),
        jax.ShapeDtypeStruct((N_V, DIM, DIM), jnp.float32),
    ),
    mesh=plsc.VectorSubcoreMesh(core_axis_name="c", subcore_axis_name="s"),
    compiler_params=pltpu.CompilerParams(use_tc_tiling_on_sc=False),
    scratch_types=[
        pltpu.VMEM((PADS, WIDTH), jnp.float32),
        pltpu.SemaphoreType.DMA,
    ],
)
def _sc_expand(shifted_hbm, q_hbm, k_hbm, v_hbm, buf, sem):
    wid = lax.axis_index("s") * 2 + lax.axis_index("c")
    pltpu.sync_copy(shifted_hbm.at[wid], buf)

    @pl.when(wid < N_Q)
    def _():
        _emit_rows(q_hbm, wid, buf, sem)

    @pl.when((wid >= N_Q) & (wid < N_Q + N_K))
    def _():
        _emit_rows(k_hbm, wid - N_Q, buf, sem)

    @pl.when(wid >= N_Q + N_K)
    def _():
        _emit_rows(v_hbm, wid - (N_Q + N_K), buf, sem)


def kernel(relative, flatten_index):
    del flatten_index
    rev = relative[:, ::-1]
    shifted = jnp.stack(
        [jnp.pad(rev, ((0, 0), (p, PADS + 1 - p))) for p in range(PADS)],
        axis=1,
    )
    return _sc_expand(shifted)
